# baseline (speedup 1.0000x reference)
import functools

import jax
import jax.numpy as jnp
from jax import lax
from jax.experimental import pallas as pl
from jax.experimental.pallas import tpu as pltpu
from jax.experimental.pallas import tpu_sc as plsc

NC = 2
NS = 16
LANES = 16

ROW_BLOCK = 400
EDGE_CHUNK = 80
ZROWS = 104


def _phase_a_body(x_ref, wq_ref, bq_ref, wk_ref, bk_ref, eqh_ref, ekh_ref):
    xb = x_ref[...]
    dn = (((1,), (1,)), ((), ()))
    eqh_ref[...] = lax.dot_general(xb, wq_ref[...], dn,
                                   preferred_element_type=jnp.float32) + bq_ref[0]
    ekh_ref[...] = lax.dot_general(xb, wk_ref[...], dn,
                                   preferred_element_type=jnp.float32) + bk_ref[0]


def _phase_a(x, Wq, bq, Wk, bk):
    n, d = x.shape
    h = Wq.shape[0]
    hh = h // 2
    nb = n // ROW_BLOCK
    grid = (nb, 2)
    out_shape = [jax.ShapeDtypeStruct((2 * n, hh), jnp.float32)] * 2
    return pl.pallas_call(
        _phase_a_body,
        grid=grid,
        in_specs=[
            pl.BlockSpec((ROW_BLOCK, d), lambda i, c: (i, 0)),
            pl.BlockSpec((hh, d), lambda i, c: (c, 0)),
            pl.BlockSpec((1, 1, hh), lambda i, c: (c, 0, 0)),
            pl.BlockSpec((hh, d), lambda i, c: (c, 0)),
            pl.BlockSpec((1, 1, hh), lambda i, c: (c, 0, 0)),
        ],
        out_specs=[
            pl.BlockSpec((ROW_BLOCK, hh), lambda i, c, nb=nb: (c * nb + i, 0)),
            pl.BlockSpec((ROW_BLOCK, hh), lambda i, c, nb=nb: (c * nb + i, 0)),
        ],
        out_shape=out_shape,
    )(x, Wq, bq.reshape(2, 1, hh), Wk, bk.reshape(2, 1, hh))


def _phase_b_body(n, e, hh, eqh_hbm, ekh_hbm, src_hbm, dst_hbm, out_hbm,
                  sidxb, didxb, didxs, didxgs, eqv, ekv, ftsh,
                  isem0, isem1, gsem0, gsem1, ssem0, ssem1):
    c = lax.axis_index("c")
    s = lax.axis_index("s")
    coff = c * n
    isem = (isem0, isem1)
    gsem = (gsem0, gsem1)
    ssem = (ssem0, ssem1)
    r0 = (n // (NS * 8)) * 8
    extra = n - NS * r0
    ept = e // NS
    nch = ept // EDGE_CHUNK
    K = EDGE_CHUNK

    zv = jnp.zeros((LANES,), jnp.float32)

    def zrow(r, carry):
        for g in range(hh // LANES):
            eqv[0, r, pl.ds(g * LANES, LANES)] = zv
        return carry

    lax.fori_loop(0, K, zrow, 0)
    nzb = r0 // K
    zrem = r0 - nzb * K
    for j in range(nzb):
        pltpu.sync_copy(eqv.at[0], ftsh.at[pl.ds(s * r0 + j * K, K), :])
    if zrem:
        pltpu.sync_copy(eqv.at[0, pl.ds(0, zrem), :],
                        ftsh.at[pl.ds(s * r0 + nzb * K, zrem), :])

    @pl.when(s == NS - 1)
    def _zero_tail():
        pltpu.sync_copy(eqv.at[0, pl.ds(0, extra), :],
                        ftsh.at[pl.ds(NS * r0, extra), :])

    plsc.subcore_barrier()

    def i_issue(i, b):
        base = s * ept + i * K
        pltpu.async_copy(src_hbm.at[pl.ds(base, K)], sidxb.at[b], isem[b])
        pltpu.async_copy(dst_hbm.at[pl.ds(base, K)], didxb.at[b], isem[b])

    def i_wait(i, b):
        base = s * ept + i * K
        pltpu.make_async_copy(src_hbm.at[pl.ds(base, K)], sidxb.at[b], isem[b]).wait()
        pltpu.make_async_copy(dst_hbm.at[pl.ds(base, K)], didxb.at[b], isem[b]).wait()

    def idx_prep(b):
        for j in range(K // LANES):
            sl = pl.ds(j * LANES, LANES)
            d = didxb[b, sl]
            sidxb[b, sl] = sidxb[b, sl] + coff
            didxgs[b, sl] = d + coff
            didxs[b, sl] = d

    def g_issue(b):
        pltpu.async_copy(ekh_hbm.at[sidxb.at[b]], ekv.at[b], gsem[b])
        pltpu.async_copy(eqh_hbm.at[didxgs.at[b]], eqv.at[b], gsem[b])

    def g_wait(b):
        pltpu.make_async_copy(ekh_hbm.at[sidxb.at[b]], ekv.at[b], gsem[b]).wait()
        pltpu.make_async_copy(eqh_hbm.at[didxgs.at[b]], eqv.at[b], gsem[b]).wait()

    def s_wait(b):
        pltpu.make_async_copy(ekv.at[b], ftsh.at[didxs.at[b]], ssem[b]).wait()

    def s_issue(b):
        pltpu.async_copy(ekv.at[b], ftsh.at[didxs.at[b]], ssem[b], add=True)

    def compute(b):
        @plsc.parallel_loop(0, K, 1, unroll=4)
        def crow(r):
            for g in range(hh // LANES):
                sl = pl.ds(g * LANES, LANES)
                ekv[b, r, sl] = jnp.maximum(eqv[b, r, sl] + ekv[b, r, sl], 0.0)

    i_issue(0, 0)
    i_wait(0, 0)
    idx_prep(0)
    g_issue(0)

    def pair(i2, carry):
        for b in range(2):
            i = 2 * i2 + b
            nxt = 1 - b

            @pl.when(i + 1 < nch)
            def _ii():
                i_issue(i + 1, nxt)

            g_wait(b)
            compute(b)

            @pl.when(i > 0)
            def _ws():
                s_wait(nxt)

            @pl.when(i + 1 < nch)
            def _gi():
                i_wait(i + 1, nxt)
                idx_prep(nxt)
                g_issue(nxt)

            s_issue(b)
        return carry

    lax.fori_loop(0, (nch - 1) // 2, pair, 0)
    ilast = nch - 1
    g_wait(0)
    compute(0)
    s_wait(1)
    s_issue(0)
    s_wait(0)
    plsc.subcore_barrier()

    @pl.when(s < NS - 1)
    def _copy_body():
        pltpu.sync_copy(ftsh.at[pl.ds(s * r0, r0), :],
                        out_hbm.at[pl.ds(coff + s * r0, r0), :])

    @pl.when(s == NS - 1)
    def _copy_tail():
        pltpu.sync_copy(ftsh.at[pl.ds((NS - 1) * r0, r0 + extra), :],
                        out_hbm.at[pl.ds(coff + (NS - 1) * r0, r0 + extra), :])


def _phase_b(eqh, ekh, src, dst):
    n2, hh = eqh.shape
    n = n2 // 2
    e = src.shape[0]
    ept = e // NS
    mesh = plsc.VectorSubcoreMesh(core_axis_name="c", subcore_axis_name="s",
                                  num_cores=NC, num_subcores=NS)
    kern = pl.kernel(
        functools.partial(_phase_b_body, n, e, hh),
        out_type=jax.ShapeDtypeStruct((2 * n, hh), jnp.float32),
        mesh=mesh,
        scratch_types=[
            pltpu.VMEM((2, EDGE_CHUNK), jnp.int32),
            pltpu.VMEM((2, EDGE_CHUNK), jnp.int32),
            pltpu.VMEM((2, EDGE_CHUNK), jnp.int32),
            pltpu.VMEM((2, EDGE_CHUNK), jnp.int32),
            pltpu.VMEM((2, EDGE_CHUNK, hh), jnp.float32),
            pltpu.VMEM((2, EDGE_CHUNK, hh), jnp.float32),
            pltpu.VMEM_SHARED((n, hh), jnp.float32),
            pltpu.SemaphoreType.DMA,
            pltpu.SemaphoreType.DMA,
            pltpu.SemaphoreType.DMA,
            pltpu.SemaphoreType.DMA,
            pltpu.SemaphoreType.DMA,
            pltpu.SemaphoreType.DMA,
        ],
    )
    return kern(eqh, ekh, src, dst)


def _phase_c_body(ft0_ref, ft1_ref, wr_ref, br_ref, out_ref):
    hh = ft0_ref.shape[1]
    dn = (((1,), (1,)), ((), ()))
    wr = wr_ref[...]
    acc = lax.dot_general(ft0_ref[...], wr[:, :hh], dn,
                          preferred_element_type=jnp.float32)
    acc = acc + lax.dot_general(ft1_ref[...], wr[:, hh:], dn,
                                preferred_element_type=jnp.float32)
    out_ref[...] = acc + br_ref[...]


def _phase_c(fth, Wr, br):
    n2, hh = fth.shape
    n = n2 // 2
    o = Wr.shape[0]
    nb = n // ROW_BLOCK
    return pl.pallas_call(
        _phase_c_body,
        grid=(nb,),
        in_specs=[
            pl.BlockSpec((ROW_BLOCK, hh), lambda i: (i, 0)),
            pl.BlockSpec((ROW_BLOCK, hh), lambda i, nb=nb: (nb + i, 0)),
            pl.BlockSpec((o, 2 * hh), lambda i: (0, 0)),
            pl.BlockSpec((1, o), lambda i: (0, 0)),
        ],
        out_specs=pl.BlockSpec((ROW_BLOCK, o), lambda i: (i, 0)),
        out_shape=jax.ShapeDtypeStruct((n, o), jnp.float32),
    )(fth, fth, Wr, br.reshape(1, o))


def kernel(x, edge_index, Wq, bq, Wk, bk, Wr, br):
    src = edge_index[0]
    dst = edge_index[1]
    eqh, ekh = _phase_a(x, Wq, bq, Wk, bk)
    fth = _phase_b(eqh, ekh, src, dst)
    return _phase_c(fth, Wr, br)


# speedup vs baseline: 1.2332x; 1.2332x over previous
import functools

import jax
import jax.numpy as jnp
import numpy as np
from jax import lax
from jax.experimental import pallas as pl
from jax.experimental.pallas import tpu as pltpu
from jax.experimental.pallas import tpu_sc as plsc

NC = 2
NS = 16
LANES = 16

ROW_BLOCK = 400
EDGE_CHUNK = 80
ZROWS = 104


def _rnd16(x):
    b = lax.bitcast_convert_type(x, jnp.uint32)
    r = b + jnp.uint32(0x7FFF) + ((b >> 16) & jnp.uint32(1))
    return r & jnp.uint32(0xFFFF0000)


def _pack_rows(x, hq):
    lo = _rnd16(x[:, :hq]) >> 16
    hi = _rnd16(x[:, hq:])
    return lax.bitcast_convert_type(lo | hi, jnp.int32)


def _phase_a_body(x_ref, wq_ref, bq_ref, wk_ref, bk_ref, eqh_ref, ekh_ref):
    xb = x_ref[...]
    dn = (((1,), (1,)), ((), ()))
    eq = lax.dot_general(xb, wq_ref[...], dn,
                         preferred_element_type=jnp.float32) + bq_ref[0]
    ek = lax.dot_general(xb, wk_ref[...], dn,
                         preferred_element_type=jnp.float32) + bk_ref[0]
    hq = eq.shape[1] // 2
    eqh_ref[...] = _pack_rows(eq, hq)
    ekh_ref[...] = _pack_rows(ek, hq)


def _phase_a(x, Wq, bq, Wk, bk):
    n, d = x.shape
    h = Wq.shape[0]
    hh = h // 2
    nb = n // ROW_BLOCK
    grid = (nb, 2)
    out_shape = [jax.ShapeDtypeStruct((2 * n, hh // 2), jnp.int32)] * 2
    return pl.pallas_call(
        _phase_a_body,
        grid=grid,
        in_specs=[
            pl.BlockSpec((ROW_BLOCK, d), lambda i, c: (i, 0)),
            pl.BlockSpec((hh, d), lambda i, c: (c, 0)),
            pl.BlockSpec((1, 1, hh), lambda i, c: (c, 0, 0)),
            pl.BlockSpec((hh, d), lambda i, c: (c, 0)),
            pl.BlockSpec((1, 1, hh), lambda i, c: (c, 0, 0)),
        ],
        out_specs=[
            pl.BlockSpec((ROW_BLOCK, hh // 2), lambda i, c, nb=nb: (c * nb + i, 0)),
            pl.BlockSpec((ROW_BLOCK, hh // 2), lambda i, c, nb=nb: (c * nb + i, 0)),
        ],
        out_shape=out_shape,
    )(x, Wq, bq.reshape(2, 1, hh), Wk, bk.reshape(2, 1, hh))


def _phase_b_body(n, e, hh, eqh_hbm, ekh_hbm, src_hbm, dst_hbm, out_hbm,
                  sidxb, didxb, didxs, didxgs, eqv, ekv, mv, ftsh,
                  isem0, isem1, gsem0, gsem1, ssem0, ssem1):
    c = lax.axis_index("c")
    s = lax.axis_index("s")
    coff = c * n
    isem = (isem0, isem1)
    gsem = (gsem0, gsem1)
    ssem = (ssem0, ssem1)
    r0 = (n // (NS * 8)) * 8
    extra = n - NS * r0
    ept = e // NS
    nch = ept // EDGE_CHUNK
    K = EDGE_CHUNK

    zv = jnp.zeros((LANES,), jnp.float32)

    def zrow(r, carry):
        for g in range(hh // LANES):
            mv[0, r, pl.ds(g * LANES, LANES)] = zv
        return carry

    lax.fori_loop(0, K, zrow, 0)
    nzb = r0 // K
    zrem = r0 - nzb * K
    for j in range(nzb):
        pltpu.sync_copy(mv.at[0], ftsh.at[pl.ds(s * r0 + j * K, K), :])
    if zrem:
        pltpu.sync_copy(mv.at[0, pl.ds(0, zrem), :],
                        ftsh.at[pl.ds(s * r0 + nzb * K, zrem), :])

    @pl.when(s == NS - 1)
    def _zero_tail():
        pltpu.sync_copy(mv.at[0, pl.ds(0, extra), :],
                        ftsh.at[pl.ds(NS * r0, extra), :])

    plsc.subcore_barrier()

    def i_issue(i, b):
        base = s * ept + i * K
        pltpu.async_copy(src_hbm.at[pl.ds(base, K)], sidxb.at[b], isem[b])
        pltpu.async_copy(dst_hbm.at[pl.ds(base, K)], didxb.at[b], isem[b])

    def i_wait(i, b):
        base = s * ept + i * K
        pltpu.make_async_copy(src_hbm.at[pl.ds(base, K)], sidxb.at[b], isem[b]).wait()
        pltpu.make_async_copy(dst_hbm.at[pl.ds(base, K)], didxb.at[b], isem[b]).wait()

    def idx_prep(b):
        for j in range(K // LANES):
            sl = pl.ds(j * LANES, LANES)
            d = didxb[b, sl]
            sidxb[b, sl] = sidxb[b, sl] + coff
            didxgs[b, sl] = d + coff
            didxs[b, sl] = d

    def g_issue(b):
        pltpu.async_copy(ekh_hbm.at[sidxb.at[b]], ekv.at[b], gsem[b])
        pltpu.async_copy(eqh_hbm.at[didxgs.at[b]], eqv.at[b], gsem[b])

    def g_wait(b):
        pltpu.make_async_copy(ekh_hbm.at[sidxb.at[b]], ekv.at[b], gsem[b]).wait()
        pltpu.make_async_copy(eqh_hbm.at[didxgs.at[b]], eqv.at[b], gsem[b]).wait()

    def s_wait(b):
        pltpu.make_async_copy(mv.at[b], ftsh.at[didxs.at[b]], ssem[b]).wait()

    def s_issue(b):
        pltpu.async_copy(mv.at[b], ftsh.at[didxs.at[b]], ssem[b], add=True)

    msk = jnp.int32(-65536)
    hw = hh // 2

    def compute(b):
        @plsc.parallel_loop(0, K, 1, unroll=4)
        def crow(r):
            for g in range(hw // LANES):
                sl = pl.ds(g * LANES, LANES)
                wq = eqv[b, r, sl]
                wk = ekv[b, r, sl]
                lo = (lax.bitcast_convert_type(wq << 16, jnp.float32)
                      + lax.bitcast_convert_type(wk << 16, jnp.float32))
                hi = (lax.bitcast_convert_type(wq & msk, jnp.float32)
                      + lax.bitcast_convert_type(wk & msk, jnp.float32))
                mv[b, r, pl.ds(g * LANES, LANES)] = jnp.maximum(lo, 0.0)
                mv[b, r, pl.ds(hw + g * LANES, LANES)] = jnp.maximum(hi, 0.0)

    i_issue(0, 0)
    i_wait(0, 0)
    idx_prep(0)
    g_issue(0)

    def pair(i2, carry):
        for b in range(2):
            i = 2 * i2 + b
            nxt = 1 - b

            @pl.when(i + 1 < nch)
            def _ii():
                i_issue(i + 1, nxt)

            g_wait(b)
            compute(b)

            @pl.when(i > 0)
            def _ws():
                s_wait(nxt)

            @pl.when(i + 1 < nch)
            def _gi():
                i_wait(i + 1, nxt)
                idx_prep(nxt)
                g_issue(nxt)

            s_issue(b)
        return carry

    lax.fori_loop(0, (nch - 1) // 2, pair, 0)
    ilast = nch - 1
    g_wait(0)
    compute(0)
    s_wait(1)
    s_issue(0)
    s_wait(0)
    plsc.subcore_barrier()

    @pl.when(s < NS - 1)
    def _copy_body():
        pltpu.sync_copy(ftsh.at[pl.ds(s * r0, r0), :],
                        out_hbm.at[pl.ds(coff + s * r0, r0), :])

    @pl.when(s == NS - 1)
    def _copy_tail():
        pltpu.sync_copy(ftsh.at[pl.ds((NS - 1) * r0, r0 + extra), :],
                        out_hbm.at[pl.ds(coff + (NS - 1) * r0, r0 + extra), :])


def _phase_b(eqh, ekh, src, dst):
    n2, hw = eqh.shape
    hh = 2 * hw
    n = n2 // 2
    e = src.shape[0]
    ept = e // NS
    mesh = plsc.VectorSubcoreMesh(core_axis_name="c", subcore_axis_name="s",
                                  num_cores=NC, num_subcores=NS)
    kern = pl.kernel(
        functools.partial(_phase_b_body, n, e, hh),
        out_type=jax.ShapeDtypeStruct((2 * n, hh), jnp.float32),
        mesh=mesh,
        compiler_params=pltpu.CompilerParams(use_tc_tiling_on_sc=False),
        scratch_types=[
            pltpu.VMEM((2, EDGE_CHUNK), jnp.int32),
            pltpu.VMEM((2, EDGE_CHUNK), jnp.int32),
            pltpu.VMEM((2, EDGE_CHUNK), jnp.int32),
            pltpu.VMEM((2, EDGE_CHUNK), jnp.int32),
            pltpu.VMEM((2, EDGE_CHUNK, hh // 2), jnp.int32),
            pltpu.VMEM((2, EDGE_CHUNK, hh // 2), jnp.int32),
            pltpu.VMEM((2, EDGE_CHUNK, hh), jnp.float32),
            pltpu.VMEM_SHARED((n, hh), jnp.float32),
            pltpu.SemaphoreType.DMA,
            pltpu.SemaphoreType.DMA,
            pltpu.SemaphoreType.DMA,
            pltpu.SemaphoreType.DMA,
            pltpu.SemaphoreType.DMA,
            pltpu.SemaphoreType.DMA,
        ],
    )
    return kern(eqh, ekh, src, dst)


def _phase_c_body(ft0_ref, ft1_ref, wr_ref, br_ref, out_ref):
    hh = ft0_ref.shape[1]
    dn = (((1,), (1,)), ((), ()))
    wr = wr_ref[...]
    acc = lax.dot_general(ft0_ref[...], wr[:, :hh], dn,
                          preferred_element_type=jnp.float32)
    acc = acc + lax.dot_general(ft1_ref[...], wr[:, hh:], dn,
                                preferred_element_type=jnp.float32)
    out_ref[...] = acc + br_ref[...]


def _phase_c(fth, Wr, br):
    n2, hh = fth.shape
    n = n2 // 2
    o = Wr.shape[0]
    nb = n // ROW_BLOCK
    return pl.pallas_call(
        _phase_c_body,
        grid=(nb,),
        in_specs=[
            pl.BlockSpec((ROW_BLOCK, hh), lambda i: (i, 0)),
            pl.BlockSpec((ROW_BLOCK, hh), lambda i, nb=nb: (nb + i, 0)),
            pl.BlockSpec((o, 2 * hh), lambda i: (0, 0)),
            pl.BlockSpec((1, o), lambda i: (0, 0)),
        ],
        out_specs=pl.BlockSpec((ROW_BLOCK, o), lambda i: (i, 0)),
        out_shape=jax.ShapeDtypeStruct((n, o), jnp.float32),
    )(fth, fth, Wr, br.reshape(1, o))


def kernel(x, edge_index, Wq, bq, Wk, bk, Wr, br):
    src = edge_index[0]
    dst = edge_index[1]
    eqh, ekh = _phase_a(x, Wq, bq, Wk, bk)
    fth = _phase_b(eqh, ekh, src, dst)
    return _phase_c(fth, Wr, br)


# speedup vs baseline: 1.3371x; 1.0843x over previous
import functools

import jax
import jax.numpy as jnp
import numpy as np
from jax import lax
from jax.experimental import pallas as pl
from jax.experimental.pallas import tpu as pltpu
from jax.experimental.pallas import tpu_sc as plsc

NC = 2
NS = 16
LANES = 16

ROW_BLOCK = 1000
EDGE_CHUNK = 80
ZROWS = 104


def _rnd16(x):
    b = lax.bitcast_convert_type(x, jnp.uint32)
    r = b + jnp.uint32(0x7FFF) + ((b >> 16) & jnp.uint32(1))
    return r & jnp.uint32(0xFFFF0000)


def _pack_rows(x, hq):
    lo = _rnd16(x[:, :hq]) >> 16
    hi = _rnd16(x[:, hq:])
    return lax.bitcast_convert_type(lo | hi, jnp.int32)


def _phase_a_body(x_ref, wq_ref, bq_ref, wk_ref, bk_ref, eqh_ref, ekh_ref):
    xb = x_ref[...]
    dn = (((1,), (1,)), ((), ()))
    eq = lax.dot_general(xb, wq_ref[...], dn,
                         preferred_element_type=jnp.float32) + bq_ref[0]
    ek = lax.dot_general(xb, wk_ref[...], dn,
                         preferred_element_type=jnp.float32) + bk_ref[0]
    hq = eq.shape[1] // 2
    eqh_ref[...] = _pack_rows(eq, hq)
    ekh_ref[...] = _pack_rows(ek, hq)


def _phase_a(x, Wq, bq, Wk, bk):
    n, d = x.shape
    h = Wq.shape[0]
    hh = h // 2
    nb = n // ROW_BLOCK
    grid = (nb, 2)
    out_shape = [jax.ShapeDtypeStruct((2 * n, hh // 2), jnp.int32)] * 2
    return pl.pallas_call(
        _phase_a_body,
        grid=grid,
        in_specs=[
            pl.BlockSpec((ROW_BLOCK, d), lambda i, c: (i, 0)),
            pl.BlockSpec((hh, d), lambda i, c: (c, 0)),
            pl.BlockSpec((1, 1, hh), lambda i, c: (c, 0, 0)),
            pl.BlockSpec((hh, d), lambda i, c: (c, 0)),
            pl.BlockSpec((1, 1, hh), lambda i, c: (c, 0, 0)),
        ],
        out_specs=[
            pl.BlockSpec((ROW_BLOCK, hh // 2), lambda i, c, nb=nb: (c * nb + i, 0)),
            pl.BlockSpec((ROW_BLOCK, hh // 2), lambda i, c, nb=nb: (c * nb + i, 0)),
        ],
        out_shape=out_shape,
    )(x, Wq, bq.reshape(2, 1, hh), Wk, bk.reshape(2, 1, hh))


def _phase_b_body(n, e, hh, eqh_hbm, ekh_hbm, src_hbm, dst_hbm, out_hbm,
                  sidxb, didxb, didxs, didxgs, eqv, ekv, mv, ftsh,
                  isem0, isem1, gsem0, gsem1, ssem0, ssem1):
    c = lax.axis_index("c")
    s = lax.axis_index("s")
    coff = c * n
    isem = (isem0, isem1)
    gsem = (gsem0, gsem1)
    ssem = (ssem0, ssem1)
    r0 = (n // (NS * 8)) * 8
    extra = n - NS * r0
    ept = e // NS
    nch = ept // EDGE_CHUNK
    K = EDGE_CHUNK

    zv = jnp.zeros((LANES,), jnp.float32)

    def zrow(r, carry):
        for g in range(hh // LANES):
            mv[0, r, pl.ds(g * LANES, LANES)] = zv
        return carry

    lax.fori_loop(0, K, zrow, 0)
    nzb = r0 // K
    zrem = r0 - nzb * K
    for j in range(nzb):
        pltpu.sync_copy(mv.at[0], ftsh.at[pl.ds(s * r0 + j * K, K), :])
    if zrem:
        pltpu.sync_copy(mv.at[0, pl.ds(0, zrem), :],
                        ftsh.at[pl.ds(s * r0 + nzb * K, zrem), :])

    @pl.when(s == NS - 1)
    def _zero_tail():
        pltpu.sync_copy(mv.at[0, pl.ds(0, extra), :],
                        ftsh.at[pl.ds(NS * r0, extra), :])

    plsc.subcore_barrier()

    def i_issue(i, b):
        base = s * ept + i * K
        pltpu.async_copy(src_hbm.at[pl.ds(base, K)], sidxb.at[b], isem[b])
        pltpu.async_copy(dst_hbm.at[pl.ds(base, K)], didxb.at[b], isem[b])

    def i_wait(i, b):
        base = s * ept + i * K
        pltpu.make_async_copy(src_hbm.at[pl.ds(base, K)], sidxb.at[b], isem[b]).wait()
        pltpu.make_async_copy(dst_hbm.at[pl.ds(base, K)], didxb.at[b], isem[b]).wait()

    def idx_prep_gather(b):
        for j in range(K // LANES):
            sl = pl.ds(j * LANES, LANES)
            sidxb[b, sl] = sidxb[b, sl] + coff
            didxgs[b, sl] = didxb[b, sl] + coff

    def idx_prep_scatter(b):
        for j in range(K // LANES):
            sl = pl.ds(j * LANES, LANES)
            didxs[b, sl] = didxb[b, sl]

    def g_issue(b):
        pltpu.async_copy(ekh_hbm.at[sidxb.at[b]], ekv.at[b], gsem[b])
        pltpu.async_copy(eqh_hbm.at[didxgs.at[b]], eqv.at[b], gsem[b])

    def g_wait(b):
        pltpu.make_async_copy(ekh_hbm.at[sidxb.at[b]], ekv.at[b], gsem[b]).wait()
        pltpu.make_async_copy(eqh_hbm.at[didxgs.at[b]], eqv.at[b], gsem[b]).wait()

    def s_wait(b):
        pltpu.make_async_copy(mv.at[b], ftsh.at[didxs.at[b]], ssem[b]).wait()

    def s_issue(b):
        pltpu.async_copy(mv.at[b], ftsh.at[didxs.at[b]], ssem[b], add=True)

    msk = jnp.int32(-65536)
    hw = hh // 2

    def compute(b):
        @plsc.parallel_loop(0, K, 1, unroll=4)
        def crow(r):
            for g in range(hw // LANES):
                sl = pl.ds(g * LANES, LANES)
                wq = eqv[b, r, sl]
                wk = ekv[b, r, sl]
                lo = (lax.bitcast_convert_type(wq << 16, jnp.float32)
                      + lax.bitcast_convert_type(wk << 16, jnp.float32))
                hi = (lax.bitcast_convert_type(wq & msk, jnp.float32)
                      + lax.bitcast_convert_type(wk & msk, jnp.float32))
                mv[b, r, pl.ds(g * LANES, LANES)] = jnp.maximum(lo, 0.0)
                mv[b, r, pl.ds(hw + g * LANES, LANES)] = jnp.maximum(hi, 0.0)

    i_issue(0, 0)
    i_wait(0, 0)
    idx_prep_gather(0)
    idx_prep_scatter(0)
    g_issue(0)

    def pair(i2, carry):
        for b in range(2):
            i = 2 * i2 + b
            nxt = 1 - b

            @pl.when(i + 1 < nch)
            def _ii():
                i_issue(i + 1, nxt)

            g_wait(b)
            compute(b)

            @pl.when(i + 1 < nch)
            def _gi():
                i_wait(i + 1, nxt)
                idx_prep_gather(nxt)
                g_issue(nxt)

            @pl.when(i > 0)
            def _ws():
                s_wait(nxt)

            @pl.when(i + 1 < nch)
            def _sp():
                idx_prep_scatter(nxt)

            s_issue(b)
        return carry

    lax.fori_loop(0, (nch - 1) // 2, pair, 0)
    ilast = nch - 1
    g_wait(0)
    compute(0)
    s_wait(1)
    s_issue(0)
    s_wait(0)
    plsc.subcore_barrier()

    @pl.when(s < NS - 1)
    def _copy_body():
        pltpu.sync_copy(ftsh.at[pl.ds(s * r0, r0), :],
                        out_hbm.at[pl.ds(coff + s * r0, r0), :])

    @pl.when(s == NS - 1)
    def _copy_tail():
        pltpu.sync_copy(ftsh.at[pl.ds((NS - 1) * r0, r0 + extra), :],
                        out_hbm.at[pl.ds(coff + (NS - 1) * r0, r0 + extra), :])


def _phase_b(eqh, ekh, src, dst):
    n2, hw = eqh.shape
    hh = 2 * hw
    n = n2 // 2
    e = src.shape[0]
    ept = e // NS
    mesh = plsc.VectorSubcoreMesh(core_axis_name="c", subcore_axis_name="s",
                                  num_cores=NC, num_subcores=NS)
    kern = pl.kernel(
        functools.partial(_phase_b_body, n, e, hh),
        out_type=jax.ShapeDtypeStruct((2 * n, hh), jnp.float32),
        mesh=mesh,
        compiler_params=pltpu.CompilerParams(use_tc_tiling_on_sc=False),
        scratch_types=[
            pltpu.VMEM((2, EDGE_CHUNK), jnp.int32),
            pltpu.VMEM((2, EDGE_CHUNK), jnp.int32),
            pltpu.VMEM((2, EDGE_CHUNK), jnp.int32),
            pltpu.VMEM((2, EDGE_CHUNK), jnp.int32),
            pltpu.VMEM((2, EDGE_CHUNK, hh // 2), jnp.int32),
            pltpu.VMEM((2, EDGE_CHUNK, hh // 2), jnp.int32),
            pltpu.VMEM((2, EDGE_CHUNK, hh), jnp.float32),
            pltpu.VMEM_SHARED((n, hh), jnp.float32),
            pltpu.SemaphoreType.DMA,
            pltpu.SemaphoreType.DMA,
            pltpu.SemaphoreType.DMA,
            pltpu.SemaphoreType.DMA,
            pltpu.SemaphoreType.DMA,
            pltpu.SemaphoreType.DMA,
        ],
    )
    return kern(eqh, ekh, src, dst)


def _phase_c_body(ft0_ref, ft1_ref, wr_ref, br_ref, out_ref):
    hh = ft0_ref.shape[1]
    dn = (((1,), (1,)), ((), ()))
    wr = wr_ref[...]
    acc = lax.dot_general(ft0_ref[...], wr[:, :hh], dn,
                          preferred_element_type=jnp.float32)
    acc = acc + lax.dot_general(ft1_ref[...], wr[:, hh:], dn,
                                preferred_element_type=jnp.float32)
    out_ref[...] = acc + br_ref[...]


def _phase_c(fth, Wr, br):
    n2, hh = fth.shape
    n = n2 // 2
    o = Wr.shape[0]
    nb = n // ROW_BLOCK
    return pl.pallas_call(
        _phase_c_body,
        grid=(nb,),
        in_specs=[
            pl.BlockSpec((ROW_BLOCK, hh), lambda i: (i, 0)),
            pl.BlockSpec((ROW_BLOCK, hh), lambda i, nb=nb: (nb + i, 0)),
            pl.BlockSpec((o, 2 * hh), lambda i: (0, 0)),
            pl.BlockSpec((1, o), lambda i: (0, 0)),
        ],
        out_specs=pl.BlockSpec((ROW_BLOCK, o), lambda i: (i, 0)),
        out_shape=jax.ShapeDtypeStruct((n, o), jnp.float32),
    )(fth, fth, Wr, br.reshape(1, o))


def kernel(x, edge_index, Wq, bq, Wk, bk, Wr, br):
    src = edge_index[0]
    dst = edge_index[1]
    eqh, ekh = _phase_a(x, Wq, bq, Wk, bk)
    fth = _phase_b(eqh, ekh, src, dst)
    return _phase_c(fth, Wr, br)


# speedup vs baseline: 1.3582x; 1.0157x over previous
import functools

import jax
import jax.numpy as jnp
import numpy as np
from jax import lax
from jax.experimental import pallas as pl
from jax.experimental.pallas import tpu as pltpu
from jax.experimental.pallas import tpu_sc as plsc

NC = 2
NS = 16
LANES = 16

ROW_BLOCK = 1000
EDGE_CHUNK = 80
ZROWS = 104


def _rnd16(x):
    b = lax.bitcast_convert_type(x, jnp.uint32)
    r = b + jnp.uint32(0x7FFF) + ((b >> 16) & jnp.uint32(1))
    return r & jnp.uint32(0xFFFF0000)


def _pack_rows(x, hq):
    lo = _rnd16(x[:, :hq]) >> 16
    hi = _rnd16(x[:, hq:])
    return lax.bitcast_convert_type(lo | hi, jnp.int32)


def _phase_a_body(x_ref, wq_ref, bq_ref, wk_ref, bk_ref, eqh_ref, ekh_ref):
    xb = x_ref[...]
    dn = (((1,), (1,)), ((), ()))
    eq = lax.dot_general(xb, wq_ref[...], dn,
                         preferred_element_type=jnp.float32) + bq_ref[0]
    ek = lax.dot_general(xb, wk_ref[...], dn,
                         preferred_element_type=jnp.float32) + bk_ref[0]
    hq = eq.shape[1] // 2
    eqh_ref[...] = _pack_rows(eq, hq)
    ekh_ref[...] = _pack_rows(ek, hq)


def _phase_a(x, Wq, bq, Wk, bk):
    n, d = x.shape
    h = Wq.shape[0]
    hh = h // 2
    nb = n // ROW_BLOCK
    grid = (nb, 2)
    out_shape = [jax.ShapeDtypeStruct((2 * n, hh // 2), jnp.int32)] * 2
    return pl.pallas_call(
        _phase_a_body,
        grid=grid,
        in_specs=[
            pl.BlockSpec((ROW_BLOCK, d), lambda i, c: (i, 0)),
            pl.BlockSpec((hh, d), lambda i, c: (c, 0)),
            pl.BlockSpec((1, 1, hh), lambda i, c: (c, 0, 0)),
            pl.BlockSpec((hh, d), lambda i, c: (c, 0)),
            pl.BlockSpec((1, 1, hh), lambda i, c: (c, 0, 0)),
        ],
        out_specs=[
            pl.BlockSpec((ROW_BLOCK, hh // 2), lambda i, c, nb=nb: (c * nb + i, 0)),
            pl.BlockSpec((ROW_BLOCK, hh // 2), lambda i, c, nb=nb: (c * nb + i, 0)),
        ],
        out_shape=out_shape,
    )(x, Wq, bq.reshape(2, 1, hh), Wk, bk.reshape(2, 1, hh))


def _phase_b_body(n, e, hh, eqh_hbm, ekh_hbm, eidx_hbm, out_hbm,
                  eidxb, sidxg, didxs, didxgs, eqv, ekv, mv, ftsh,
                  isem0, isem1, gsem0, gsem1, ssem0, ssem1):
    c = lax.axis_index("c")
    s = lax.axis_index("s")
    coff = c * n
    isem = (isem0, isem1)
    gsem = (gsem0, gsem1)
    ssem = (ssem0, ssem1)
    r0 = (n // (NS * 8)) * 8
    extra = n - NS * r0
    ept = e // NS
    nch = ept // EDGE_CHUNK
    K = EDGE_CHUNK

    zv = jnp.zeros((LANES,), jnp.float32)

    def zrow(r, carry):
        for g in range(hh // LANES):
            mv[0, r, pl.ds(g * LANES, LANES)] = zv
        return carry

    lax.fori_loop(0, K, zrow, 0)
    nzb = r0 // K
    zrem = r0 - nzb * K
    for j in range(nzb):
        pltpu.sync_copy(mv.at[0], ftsh.at[pl.ds(s * r0 + j * K, K), :])
    if zrem:
        pltpu.sync_copy(mv.at[0, pl.ds(0, zrem), :],
                        ftsh.at[pl.ds(s * r0 + nzb * K, zrem), :])

    @pl.when(s == NS - 1)
    def _zero_tail():
        pltpu.sync_copy(mv.at[0, pl.ds(0, extra), :],
                        ftsh.at[pl.ds(NS * r0, extra), :])

    plsc.subcore_barrier()

    def i_issue(i, b):
        base = s * ept + i * K
        pltpu.async_copy(eidx_hbm.at[:, pl.ds(base, K)], eidxb.at[b], isem[b])

    def i_wait(i, b):
        base = s * ept + i * K
        pltpu.make_async_copy(eidx_hbm.at[:, pl.ds(base, K)], eidxb.at[b],
                              isem[b]).wait()

    def idx_prep_gather(b):
        for j in range(K // LANES):
            sl = pl.ds(j * LANES, LANES)
            sidxg[b, sl] = eidxb[b, 0, sl] + coff
            didxgs[b, sl] = eidxb[b, 1, sl] + coff

    def idx_prep_scatter(b):
        for j in range(K // LANES):
            sl = pl.ds(j * LANES, LANES)
            didxs[b, sl] = eidxb[b, 1, sl]

    def g_issue(b):
        pltpu.async_copy(ekh_hbm.at[sidxg.at[b]], ekv.at[b], gsem[b])
        pltpu.async_copy(eqh_hbm.at[didxgs.at[b]], eqv.at[b], gsem[b])

    def g_wait(b):
        pltpu.make_async_copy(ekh_hbm.at[sidxg.at[b]], ekv.at[b], gsem[b]).wait()
        pltpu.make_async_copy(eqh_hbm.at[didxgs.at[b]], eqv.at[b], gsem[b]).wait()

    def s_wait(b):
        pltpu.make_async_copy(mv.at[b], ftsh.at[didxs.at[b]], ssem[b]).wait()

    def s_issue(b):
        pltpu.async_copy(mv.at[b], ftsh.at[didxs.at[b]], ssem[b], add=True)

    msk = jnp.int32(-65536)
    hw = hh // 2

    def compute(b):
        @plsc.parallel_loop(0, K, 1, unroll=4)
        def crow(r):
            for g in range(hw // LANES):
                sl = pl.ds(g * LANES, LANES)
                wq = eqv[b, r, sl]
                wk = ekv[b, r, sl]
                lo = (lax.bitcast_convert_type(wq << 16, jnp.float32)
                      + lax.bitcast_convert_type(wk << 16, jnp.float32))
                hi = (lax.bitcast_convert_type(wq & msk, jnp.float32)
                      + lax.bitcast_convert_type(wk & msk, jnp.float32))
                mv[b, r, pl.ds(g * LANES, LANES)] = jnp.maximum(lo, 0.0)
                mv[b, r, pl.ds(hw + g * LANES, LANES)] = jnp.maximum(hi, 0.0)

    i_issue(0, 0)
    i_wait(0, 0)
    idx_prep_gather(0)
    idx_prep_scatter(0)
    g_issue(0)

    def pair(i2, carry):
        for b in range(2):
            i = 2 * i2 + b
            nxt = 1 - b

            @pl.when(i + 1 < nch)
            def _ii():
                i_issue(i + 1, nxt)

            g_wait(b)
            compute(b)

            @pl.when(i + 1 < nch)
            def _gi():
                i_wait(i + 1, nxt)
                idx_prep_gather(nxt)
                g_issue(nxt)

            @pl.when(i > 0)
            def _ws():
                s_wait(nxt)

            @pl.when(i + 1 < nch)
            def _sp():
                idx_prep_scatter(nxt)

            s_issue(b)
        return carry

    lax.fori_loop(0, (nch - 1) // 2, pair, 0)
    ilast = nch - 1
    g_wait(0)
    compute(0)
    s_wait(1)
    s_issue(0)
    s_wait(0)
    plsc.subcore_barrier()

    @pl.when(s < NS - 1)
    def _copy_body():
        pltpu.sync_copy(ftsh.at[pl.ds(s * r0, r0), :],
                        out_hbm.at[pl.ds(coff + s * r0, r0), :])

    @pl.when(s == NS - 1)
    def _copy_tail():
        pltpu.sync_copy(ftsh.at[pl.ds((NS - 1) * r0, r0 + extra), :],
                        out_hbm.at[pl.ds(coff + (NS - 1) * r0, r0 + extra), :])


def _phase_b(eqh, ekh, eidx):
    n2, hw = eqh.shape
    hh = 2 * hw
    n = n2 // 2
    e = eidx.shape[1]
    ept = e // NS
    mesh = plsc.VectorSubcoreMesh(core_axis_name="c", subcore_axis_name="s",
                                  num_cores=NC, num_subcores=NS)
    kern = pl.kernel(
        functools.partial(_phase_b_body, n, e, hh),
        out_type=jax.ShapeDtypeStruct((2 * n, hh), jnp.float32),
        mesh=mesh,
        compiler_params=pltpu.CompilerParams(use_tc_tiling_on_sc=False),
        scratch_types=[
            pltpu.VMEM((2, 2, EDGE_CHUNK), jnp.int32),
            pltpu.VMEM((2, EDGE_CHUNK), jnp.int32),
            pltpu.VMEM((2, EDGE_CHUNK), jnp.int32),
            pltpu.VMEM((2, EDGE_CHUNK), jnp.int32),
            pltpu.VMEM((2, EDGE_CHUNK, hh // 2), jnp.int32),
            pltpu.VMEM((2, EDGE_CHUNK, hh // 2), jnp.int32),
            pltpu.VMEM((2, EDGE_CHUNK, hh), jnp.float32),
            pltpu.VMEM_SHARED((n, hh), jnp.float32),
            pltpu.SemaphoreType.DMA,
            pltpu.SemaphoreType.DMA,
            pltpu.SemaphoreType.DMA,
            pltpu.SemaphoreType.DMA,
            pltpu.SemaphoreType.DMA,
            pltpu.SemaphoreType.DMA,
        ],
    )
    return kern(eqh, ekh, eidx)


def _phase_c_body(ft0_ref, ft1_ref, wr_ref, br_ref, out_ref):
    hh = ft0_ref.shape[1]
    dn = (((1,), (1,)), ((), ()))
    wr = wr_ref[...]
    acc = lax.dot_general(ft0_ref[...], wr[:, :hh], dn,
                          preferred_element_type=jnp.float32)
    acc = acc + lax.dot_general(ft1_ref[...], wr[:, hh:], dn,
                                preferred_element_type=jnp.float32)
    out_ref[...] = acc + br_ref[...]


def _phase_c(fth, Wr, br):
    n2, hh = fth.shape
    n = n2 // 2
    o = Wr.shape[0]
    nb = n // ROW_BLOCK
    return pl.pallas_call(
        _phase_c_body,
        grid=(nb,),
        in_specs=[
            pl.BlockSpec((ROW_BLOCK, hh), lambda i: (i, 0)),
            pl.BlockSpec((ROW_BLOCK, hh), lambda i, nb=nb: (nb + i, 0)),
            pl.BlockSpec((o, 2 * hh), lambda i: (0, 0)),
            pl.BlockSpec((1, o), lambda i: (0, 0)),
        ],
        out_specs=pl.BlockSpec((ROW_BLOCK, o), lambda i: (i, 0)),
        out_shape=jax.ShapeDtypeStruct((n, o), jnp.float32),
    )(fth, fth, Wr, br.reshape(1, o))


def kernel(x, edge_index, Wq, bq, Wk, bk, Wr, br):
    eqh, ekh = _phase_a(x, Wq, bq, Wk, bk)
    fth = _phase_b(eqh, ekh, edge_index)
    return _phase_c(fth, Wr, br)
